# baseline (speedup 1.0000x reference)
import jax
import jax.numpy as jnp
from jax.experimental import pallas as pl
from jax.experimental.pallas import tpu as pltpu

_N = 10000
_D = 128
_B = 64
_E = 2 * _B


def _body(ids_ref, dflat_ref, mem_ref, delta_hbm, idcol_ref, idrow_ref,
          W1s_ref, b1s_ref, W2s_ref, b2s_ref,
          W1t_ref, b1t_ref, W2t_ref, b2t_ref,
          Wih_ref, bih_ref, Whh_ref, bhh_ref,
          out_ref, gm_ref, gd_ref, nr_ref, sem):
    f32 = jnp.float32

    def g_start(k, c):
        j = dflat_ref[k]
        pltpu.make_async_copy(delta_hbm.at[pl.ds(j, 1), :],
                              gd_ref.at[pl.ds(k, 1), :], sem).start()
        return c
    jax.lax.fori_loop(0, _E, g_start, 0)

    def m_gather(k, c):
        i = ids_ref[k]
        gm_ref[pl.ds(k, 1), :] = mem_ref[pl.ds(i, 1), :]
        return c
    jax.lax.fori_loop(0, _E, m_gather, 0)

    def g_wait(k, c):
        j = dflat_ref[k]
        pltpu.make_async_copy(delta_hbm.at[pl.ds(j, 1), :],
                              gd_ref.at[pl.ds(k, 1), :], sem).wait()
        return c
    jax.lax.fori_loop(0, _E, g_wait, 0)

    gm = gm_ref[...]
    gd = gd_ref[...]

    xs = jnp.concatenate([gm[0:_B], gm[_B:_E], gd[0:_B]], axis=1)
    xt = jnp.concatenate([gm[_B:_E], gm[0:_B], gd[_B:_E]], axis=1)
    hs = jax.nn.relu(jnp.dot(xs, W1s_ref[...], preferred_element_type=f32)
                     + b1s_ref[...])
    ms = jnp.dot(hs, W2s_ref[...], preferred_element_type=f32) + b2s_ref[...]
    ht = jax.nn.relu(jnp.dot(xt, W1t_ref[...], preferred_element_type=f32)
                     + b1t_ref[...])
    mt = jnp.dot(ht, W2t_ref[...], preferred_element_type=f32) + b2t_ref[...]
    msgs = jnp.concatenate([ms, mt], axis=0)

    coll = (idcol_ref[...] == idrow_ref[...]).astype(f32)
    cnt = jnp.sum(coll, axis=1, keepdims=True)
    agg = jnp.dot(coll, msgs, preferred_element_type=f32) / cnt

    gi = jnp.dot(agg, Wih_ref[...], preferred_element_type=f32) + bih_ref[...]
    gh = jnp.dot(gm, Whh_ref[...], preferred_element_type=f32) + bhh_ref[...]
    r = jax.nn.sigmoid(gi[:, 0:_D] + gh[:, 0:_D])
    z = jax.nn.sigmoid(gi[:, _D:2 * _D] + gh[:, _D:2 * _D])
    n = jnp.tanh(gi[:, 2 * _D:3 * _D] + r * gh[:, 2 * _D:3 * _D])
    nr_ref[...] = (1.0 - z) * n + z * gm

    out_ref[...] = mem_ref[...]

    def s_body(k, c):
        i = ids_ref[k]
        out_ref[pl.ds(i, 1), :] = nr_ref[pl.ds(k, 1), :]
        return c
    jax.lax.fori_loop(0, _E, s_body, 0)


def kernel(memory, source, target, delta_t_vec,
           W_src1, b_src1, W_src2, b_src2,
           W_tar1, b_tar1, W_tar2, b_tar2,
           W_ih, W_hh, b_ih, b_hh):
    f32 = jnp.float32
    src = source[:, 0].astype(jnp.int32)
    tar = target[:, 0].astype(jnp.int32)
    ids = jnp.concatenate([src, tar])
    bidx = jnp.arange(_B, dtype=jnp.int32)
    dflat = jnp.concatenate([bidx * _N + src, bidx * _N + tar])
    delta2d = delta_t_vec.reshape(_B * _N, _D)

    vspec = pl.BlockSpec(memory_space=pltpu.MemorySpace.VMEM)
    sspec = pl.BlockSpec(memory_space=pltpu.MemorySpace.SMEM)
    call = pl.pallas_call(
        _body,
        out_shape=jax.ShapeDtypeStruct((_N, _D), f32),
        in_specs=[
            sspec,
            sspec,
            vspec,
            pl.BlockSpec(memory_space=pl.ANY),
            vspec, vspec,
            vspec, vspec, vspec, vspec,
            vspec, vspec, vspec, vspec,
            vspec, vspec, vspec, vspec,
        ],
        out_specs=vspec,
        scratch_shapes=[
            pltpu.MemorySpace.VMEM((_E, _D), f32),
            pltpu.MemorySpace.VMEM((_E, _D), f32),
            pltpu.MemorySpace.VMEM((_E, _D), f32),
            pltpu.SemaphoreType.DMA,
        ],
    )
    return call(
        ids, dflat, memory, delta2d,
        ids[:, None], ids[None, :],
        W_src1.T, b_src1[None, :], W_src2.T, b_src2[None, :],
        W_tar1.T, b_tar1[None, :], W_tar2.T, b_tar2[None, :],
        W_ih.T, b_ih[None, :], W_hh.T, b_hh[None, :],
    )
